# baseline (speedup 1.0000x reference)
import functools

import jax
import jax.numpy as jnp
from jax import lax
from jax.experimental import pallas as pl
from jax.experimental.pallas import tpu as pltpu
from jax.experimental.pallas import tpu_sc as plsc

B = 1024
L = 200
K = 8
D = 32
NCLS = 64
V = 10000
LK = L * K
GK = 13
LKP = GK * 128
XP = 256
NC = 2
NS = 16
NW = NC * NS
BPW = B // NW


def _ln_body(emb_ref, g_ref, b_ref, out_ref):
    e = emb_ref[...]
    m = jnp.mean(e, axis=-1, keepdims=True)
    var = jnp.mean((e - m) * (e - m), axis=-1, keepdims=True)
    out_ref[...] = (e - m) * lax.rsqrt(var + 1e-5) * g_ref[...] + b_ref[...]


def _fc_body(pp_ref, w_ref, b_ref, out_ref):
    pp = pp_ref[...]
    di = lax.broadcasted_iota(jnp.int32, (D * 16, D), 0)
    ei = lax.broadcasted_iota(jnp.int32, (D * 16, D), 1)
    r = jnp.where(ei == ((di // 16 + di % 16) & (D - 1)), 1.0, 0.0)
    p = lax.dot_general(pp, r, (((1,), (0,)), ((), ())),
                        preferred_element_type=jnp.float32)
    p = jnp.maximum(p, 1.0)
    out_ref[...] = lax.dot_general(
        p, w_ref[...], (((1,), (1,)), ((), ())),
        preferred_element_type=jnp.float32) + b_ref[...]


NE = (V - 1) * V + 1
NE16 = (NE - 1) // 16


def _sc_body(x_hbm, nb_hbm, wed_hbm, lnemb_hbm, we_hbm, eta_hbm, out_hbm,
             x_v, nb_v, wi_v, we_v, eta_v, rows_v, self_v, out_v, sem):
    wid = lax.axis_index("s") * NC + lax.axis_index("c")
    b0 = wid * BPW

    iota = lax.iota(jnp.int32, 16)
    zi16 = jnp.zeros((16,), jnp.int32)
    zf16 = jnp.zeros((16,), jnp.float32)
    dcols = [(iota + d) & 31 for d in range(D)]

    for i in range((XP - L) // 16 + 1):
        x_v[pl.ds(192 + 16 * i, 16)] = zi16
    for i in range((LKP - LK) // 16):
        nb_v[pl.ds(LK + 16 * i, 16)] = zi16
        wi_v[pl.ds(LK + 16 * i, 16)] = zi16

    def g_body(g, carry):
        tok = g * 16 + iota
        x16 = plsc.load_gather(x_v, [tok])
        m16 = jnp.where(x16 != 0, 1.0, 0.0)
        er16 = plsc.load_gather(eta_v, [tok])
        eta16 = 1.0 / (1.0 + jnp.exp(-er16))
        base = g * 128 + iota * 8
        cf = []
        cnt = zf16
        for k in range(K):
            idxk = base + k
            nb16 = plsc.load_gather(nb_v, [idxk])
            w16 = plsc.load_gather(we_v, [idxk])
            mk = jnp.where(nb16 != 0, 1.0, 0.0)
            cnt = cnt + mk
            cf.append(w16 * mk)
        denom = jnp.maximum(cnt, 1.0)
        a16 = m16 * (1.0 - eta16) / denom
        s16 = m16 * eta16
        cks = [cf[k] * a16 for k in range(K)]
        rws = [base + k for k in range(K)]
        for d in range(D):
            dd = dcols[d]
            v = plsc.load_gather(self_v, [tok, dd])
            acc = s16 * v
            for k in range(K):
                v = plsc.load_gather(rows_v, [rws[k], dd])
                acc = acc + cks[k] * v
            plsc.addupdate(out_v.at[pl.ds(d * 16, 16)], acc)
        return carry

    def b_body(bi, carry):
        b = b0 + bi
        h1 = pltpu.async_copy(x_hbm.at[b], x_v.at[pl.ds(0, L)], sem)
        h2 = pltpu.async_copy(nb_hbm.at[b], nb_v.at[pl.ds(0, LK)], sem)
        h3 = pltpu.async_copy(wed_hbm.at[b], wi_v.at[pl.ds(0, LK)], sem)
        with jax.named_scope("idxload"):
            h1.wait()
            h2.wait()
            h3.wait()
        with jax.named_scope("gather"):
            hs = [
                pltpu.async_copy(lnemb_hbm.at[nb_v], rows_v, sem),
                pltpu.async_copy(we_hbm.at[wi_v], we_v, sem),
                pltpu.async_copy(lnemb_hbm.at[x_v], self_v, sem),
                pltpu.async_copy(eta_hbm.at[x_v], eta_v, sem),
            ]
            for h in hs:
                h.wait()
        with jax.named_scope("compute"):
            for d in range(D):
                out_v[pl.ds(d * 16, 16)] = zf16
            lax.fori_loop(0, GK, g_body, 0)
        with jax.named_scope("writeout"):
            pltpu.sync_copy(out_v, out_hbm.at[b])
        return carry

    lax.fori_loop(0, BPW, b_body, 0)


_sc_main = functools.partial(
    pl.kernel,
    out_type=jax.ShapeDtypeStruct((B, D * 16), jnp.float32),
    mesh=plsc.VectorSubcoreMesh(core_axis_name="c", subcore_axis_name="s"),
    compiler_params=pltpu.CompilerParams(
        needs_layout_passes=False, use_tc_tiling_on_sc=False),
    scratch_types=[
        pltpu.VMEM((XP,), jnp.int32),
        pltpu.VMEM((LKP,), jnp.int32),
        pltpu.VMEM((LKP,), jnp.int32),
        pltpu.VMEM((LKP,), jnp.float32),
        pltpu.VMEM((XP,), jnp.float32),
        pltpu.VMEM((LKP, D), jnp.float32),
        pltpu.VMEM((XP, D), jnp.float32),
        pltpu.VMEM((D * 16,), jnp.float32),
        pltpu.SemaphoreType.DMA,
    ],
)(_sc_body)


def kernel(x, nb_x, w_edge, emb, we, eta_t, ln_g, ln_b, fc_w, fc_b):
    ln_emb = pl.pallas_call(
        _ln_body,
        out_shape=jax.ShapeDtypeStruct((V, D), jnp.float32),
    )(emb, ln_g.reshape(1, D), ln_b.reshape(1, D))

    pooled_part = _sc_main(
        x.astype(jnp.int32),
        nb_x.reshape(B, LK).astype(jnp.int32),
        w_edge.reshape(B, LK).astype(jnp.int32),
        ln_emb, we.reshape(-1), eta_t.reshape(-1))

    scores = pl.pallas_call(
        _fc_body,
        out_shape=jax.ShapeDtypeStruct((B, NCLS), jnp.float32),
    )(pooled_part, fc_w, fc_b.reshape(1, NCLS))
    return scores


# speedup vs baseline: 1.1145x; 1.1145x over previous
import functools

import jax
import jax.numpy as jnp
from jax import lax
from jax.experimental import pallas as pl
from jax.experimental.pallas import tpu as pltpu
from jax.experimental.pallas import tpu_sc as plsc

B = 1024
L = 200
K = 8
D = 32
NCLS = 64
V = 10000
LK = L * K
GK = 13
LKP = GK * 128
XP = 256
NC = 2
NS = 16
NW = NC * NS
BPW = B // NW


def _ln_body(emb_ref, g_ref, b_ref, out_ref):
    e = emb_ref[...]
    m = jnp.mean(e, axis=-1, keepdims=True)
    var = jnp.mean((e - m) * (e - m), axis=-1, keepdims=True)
    out_ref[...] = (e - m) * lax.rsqrt(var + 1e-5) * g_ref[...] + b_ref[...]


def _fc_body(pp_ref, w_ref, b_ref, out_ref):
    pp = pp_ref[...]
    di = lax.broadcasted_iota(jnp.int32, (D * 16, D), 0)
    ei = lax.broadcasted_iota(jnp.int32, (D * 16, D), 1)
    r = jnp.where(ei == ((di // 16 + di % 16) & (D - 1)), 1.0, 0.0)
    p = lax.dot_general(pp, r, (((1,), (0,)), ((), ())),
                        preferred_element_type=jnp.float32)
    p = jnp.maximum(p, 1.0)
    out_ref[...] = lax.dot_general(
        p, w_ref[...], (((1,), (1,)), ((), ())),
        preferred_element_type=jnp.float32) + b_ref[...]


NE = (V - 1) * V + 1
NE16 = (NE - 1) // 16


def _sc_body(x_hbm, nb_hbm, wed_hbm, lnemb_hbm, we_hbm, wtail_hbm, eta_hbm,
             out_hbm, x_v, nb_v, wi_v, rid_v, we_v, wtail_v, eta_v, rows_v,
             self_v, out_v, sem):
    wid = lax.axis_index("s") * NC + lax.axis_index("c")
    b0 = wid * BPW

    iota = lax.iota(jnp.int32, 16)
    zi16 = jnp.zeros((16,), jnp.int32)
    zf16 = jnp.zeros((16,), jnp.float32)
    dcols = [(iota + d) & 31 for d in range(D)]

    for i in range((XP - L) // 16 + 1):
        x_v[pl.ds(192 + 16 * i, 16)] = zi16
    for i in range((LKP - LK) // 16):
        nb_v[pl.ds(LK + 16 * i, 16)] = zi16
        wi_v[pl.ds(LK + 16 * i, 16)] = zi16
    pltpu.sync_copy(wtail_hbm, wtail_v)
    wt15 = plsc.load_gather(wtail_v, [iota * 0 + 15])

    def g_body(g, carry):
        tok = g * 16 + iota
        x16 = plsc.load_gather(x_v, [tok])
        m16 = jnp.where(x16 != 0, 1.0, 0.0)
        er16 = plsc.load_gather(eta_v, [tok])
        eta16 = 1.0 / (1.0 + jnp.exp(-er16))
        base = g * 128 + iota * 8
        cf = []
        cnt = zf16
        for k in range(K):
            idxk = base + k
            nb16 = plsc.load_gather(nb_v, [idxk])
            wi16 = plsc.load_gather(wi_v, [idxk])
            w16 = plsc.load_gather(we_v, [idxk, wi16 & 15])
            w16 = jnp.where(wi16 == NE - 1, wt15, w16)
            mk = jnp.where(nb16 != 0, 1.0, 0.0)
            cnt = cnt + mk
            cf.append(w16 * mk)
        denom = jnp.maximum(cnt, 1.0)
        a16 = m16 * (1.0 - eta16) / denom
        s16 = m16 * eta16
        cks = [cf[k] * a16 for k in range(K)]
        rws = [base + k for k in range(K)]
        for d in range(D):
            dd = dcols[d]
            v = plsc.load_gather(self_v, [tok, dd])
            acc = s16 * v
            for k in range(K):
                v = plsc.load_gather(rows_v, [rws[k], dd])
                acc = acc + cks[k] * v
            plsc.addupdate(out_v.at[pl.ds(d * 16, 16)], acc)
        return carry

    def b_body(bi, carry):
        b = b0 + bi
        h1 = pltpu.async_copy(x_hbm.at[b], x_v.at[pl.ds(0, L)], sem)
        h2 = pltpu.async_copy(nb_hbm.at[b], nb_v.at[pl.ds(0, LK)], sem)
        h3 = pltpu.async_copy(wed_hbm.at[b], wi_v.at[pl.ds(0, LK)], sem)
        with jax.named_scope("idxload"):
            h1.wait()
            h2.wait()
            h3.wait()
        with jax.named_scope("ridcompute"):
            for j in range(LKP // 16):
                sl = pl.ds(j * 16, 16)
                rid_v[sl] = jnp.minimum(
                    lax.shift_right_logical(wi_v[sl], 4), NE16 - 1)
        with jax.named_scope("gather"):
            hs = [
                pltpu.async_copy(lnemb_hbm.at[nb_v], rows_v, sem),
                pltpu.async_copy(we_hbm.at[rid_v], we_v, sem),
                pltpu.async_copy(lnemb_hbm.at[x_v], self_v, sem),
                pltpu.async_copy(eta_hbm.at[x_v], eta_v, sem),
            ]
            for h in hs:
                h.wait()
        with jax.named_scope("compute"):
            for d in range(D):
                out_v[pl.ds(d * 16, 16)] = zf16
            lax.fori_loop(0, GK, g_body, 0)
        with jax.named_scope("writeout"):
            pltpu.sync_copy(out_v, out_hbm.at[b])
        return carry

    lax.fori_loop(0, BPW, b_body, 0)


_sc_main = functools.partial(
    pl.kernel,
    out_type=jax.ShapeDtypeStruct((B, D * 16), jnp.float32),
    mesh=plsc.VectorSubcoreMesh(core_axis_name="c", subcore_axis_name="s"),
    compiler_params=pltpu.CompilerParams(
        needs_layout_passes=False, use_tc_tiling_on_sc=False),
    scratch_types=[
        pltpu.VMEM((XP,), jnp.int32),
        pltpu.VMEM((LKP,), jnp.int32),
        pltpu.VMEM((LKP,), jnp.int32),
        pltpu.VMEM((LKP,), jnp.int32),
        pltpu.VMEM((LKP, 16), jnp.float32),
        pltpu.VMEM((16,), jnp.float32),
        pltpu.VMEM((XP,), jnp.float32),
        pltpu.VMEM((LKP, D), jnp.float32),
        pltpu.VMEM((XP, D), jnp.float32),
        pltpu.VMEM((D * 16,), jnp.float32),
        pltpu.SemaphoreType.DMA,
    ],
)(_sc_body)


def kernel(x, nb_x, w_edge, emb, we, eta_t, ln_g, ln_b, fc_w, fc_b):
    ln_emb = pl.pallas_call(
        _ln_body,
        out_shape=jax.ShapeDtypeStruct((V, D), jnp.float32),
    )(emb, ln_g.reshape(1, D), ln_b.reshape(1, D))

    pooled_part = _sc_main(
        x.astype(jnp.int32),
        nb_x.reshape(B, LK).astype(jnp.int32),
        w_edge.reshape(B, LK).astype(jnp.int32),
        ln_emb, we[:NE - 1].reshape(NE16, 16), we[NE - 16:].reshape(16),
        eta_t.reshape(-1))

    scores = pl.pallas_call(
        _fc_body,
        out_shape=jax.ShapeDtypeStruct((B, NCLS), jnp.float32),
    )(pooled_part, fc_w, fc_b.reshape(1, NCLS))
    return scores


# speedup vs baseline: 1.1168x; 1.0021x over previous
import functools

import jax
import jax.numpy as jnp
from jax import lax
from jax.experimental import pallas as pl
from jax.experimental.pallas import tpu as pltpu
from jax.experimental.pallas import tpu_sc as plsc

B = 1024
L = 200
K = 8
D = 32
NCLS = 64
V = 10000
LK = L * K
GK = 13
LKP = GK * 128
XP = 256
NC = 2
NS = 16
NW = NC * NS
BPW = B // NW


def _ln_body(emb_ref, g_ref, b_ref, out_ref):
    e = emb_ref[...]
    m = jnp.mean(e, axis=-1, keepdims=True)
    var = jnp.mean((e - m) * (e - m), axis=-1, keepdims=True)
    out_ref[...] = (e - m) * lax.rsqrt(var + 1e-5) * g_ref[...] + b_ref[...]


def _fc_body(pp_ref, w_ref, b_ref, out_ref):
    pp = pp_ref[...]
    di = lax.broadcasted_iota(jnp.int32, (D * 16, D), 0)
    ei = lax.broadcasted_iota(jnp.int32, (D * 16, D), 1)
    r = jnp.where(ei == ((di // 16 + di % 16) & (D - 1)), 1.0, 0.0)
    p = lax.dot_general(pp, r, (((1,), (0,)), ((), ())),
                        preferred_element_type=jnp.float32)
    p = jnp.maximum(p, 1.0)
    out_ref[...] = lax.dot_general(
        p, w_ref[...], (((1,), (1,)), ((), ())),
        preferred_element_type=jnp.float32) + b_ref[...]


NE = (V - 1) * V + 1
NE16 = (NE - 1) // 16


def _sc_body(x_hbm, nb_hbm, wed_hbm, lnemb_hbm, we_hbm, wtail_hbm, eta_hbm,
             out_hbm, x_v, nb_v, wi_v, rid_v, we_v, wtail_v, eta_tbl, rows_v,
             self_v, out_v, sem):
    wid = lax.axis_index("s") * NC + lax.axis_index("c")
    b0 = wid * BPW

    iota = lax.iota(jnp.int32, 16)
    zi16 = jnp.zeros((16,), jnp.int32)
    zf16 = jnp.zeros((16,), jnp.float32)
    dcols = [(iota + d) & 31 for d in range(D)]

    for i in range((XP - L) // 16 + 1):
        x_v[pl.ds(192 + 16 * i, 16)] = zi16
    for i in range((LKP - LK) // 16):
        nb_v[pl.ds(LK + 16 * i, 16)] = zi16
        wi_v[pl.ds(LK + 16 * i, 16)] = zi16
    pltpu.sync_copy(wtail_hbm, wtail_v)
    wt15 = plsc.load_gather(wtail_v, [iota * 0 + 15])
    pltpu.sync_copy(eta_hbm, eta_tbl.at[pl.ds(0, V)])

    def g_body(g, carry):
        tok = g * 16 + iota
        x16 = plsc.load_gather(x_v, [tok])
        m16 = jnp.where(x16 != 0, 1.0, 0.0)
        er16 = plsc.load_gather(eta_tbl, [x16])
        eta16 = 1.0 / (1.0 + jnp.exp(-er16))
        base = g * 128 + iota * 8
        cf = []
        cnt = zf16
        for k in range(K):
            idxk = base + k
            nb16 = plsc.load_gather(nb_v, [idxk])
            wi16 = plsc.load_gather(wi_v, [idxk])
            w16 = plsc.load_gather(we_v, [idxk, wi16 & 15])
            w16 = jnp.where(wi16 == NE - 1, wt15, w16)
            mk = jnp.where(nb16 != 0, 1.0, 0.0)
            cnt = cnt + mk
            cf.append(w16 * mk)
        denom = jnp.maximum(cnt, 1.0)
        a16 = m16 * (1.0 - eta16) / denom
        s16 = m16 * eta16
        cks = [cf[k] * a16 for k in range(K)]
        rws = [base + k for k in range(K)]
        for d in range(D):
            dd = dcols[d]
            v = plsc.load_gather(self_v, [tok, dd])
            acc = s16 * v
            for k in range(K):
                v = plsc.load_gather(rows_v, [rws[k], dd])
                acc = acc + cks[k] * v
            plsc.addupdate(out_v.at[pl.ds(d * 16, 16)], acc)
        return carry

    def b_body(bi, carry):
        b = b0 + bi
        h1 = pltpu.async_copy(x_hbm.at[b], x_v.at[pl.ds(0, L)], sem)
        h2 = pltpu.async_copy(nb_hbm.at[b], nb_v.at[pl.ds(0, LK)], sem)
        h3 = pltpu.async_copy(wed_hbm.at[b], wi_v.at[pl.ds(0, LK)], sem)
        h1.wait()
        h2.wait()
        h3.wait()
        for j in range(LKP // 16):
            sl = pl.ds(j * 16, 16)
            rid_v[sl] = jnp.minimum(
                lax.shift_right_logical(wi_v[sl], 4), NE16 - 1)
        hs = [
            pltpu.async_copy(lnemb_hbm.at[nb_v], rows_v, sem),
            pltpu.async_copy(we_hbm.at[rid_v], we_v, sem),
            pltpu.async_copy(lnemb_hbm.at[x_v], self_v, sem),
        ]
        for h in hs:
            h.wait()
        for d in range(D):
            out_v[pl.ds(d * 16, 16)] = zf16
        lax.fori_loop(0, GK, g_body, 0)
        pltpu.sync_copy(out_v, out_hbm.at[b])
        return carry

    lax.fori_loop(0, BPW, b_body, 0)


_sc_main = functools.partial(
    pl.kernel,
    out_type=jax.ShapeDtypeStruct((B, D * 16), jnp.float32),
    mesh=plsc.VectorSubcoreMesh(core_axis_name="c", subcore_axis_name="s"),
    compiler_params=pltpu.CompilerParams(
        needs_layout_passes=False, use_tc_tiling_on_sc=False),
    scratch_types=[
        pltpu.VMEM((XP,), jnp.int32),
        pltpu.VMEM((LKP,), jnp.int32),
        pltpu.VMEM((LKP,), jnp.int32),
        pltpu.VMEM((LKP,), jnp.int32),
        pltpu.VMEM((LKP, 16), jnp.float32),
        pltpu.VMEM((16,), jnp.float32),
        pltpu.VMEM((V,), jnp.float32),
        pltpu.VMEM((LKP, D), jnp.float32),
        pltpu.VMEM((XP, D), jnp.float32),
        pltpu.VMEM((D * 16,), jnp.float32),
        pltpu.SemaphoreType.DMA,
    ],
)(_sc_body)


def kernel(x, nb_x, w_edge, emb, we, eta_t, ln_g, ln_b, fc_w, fc_b):
    ln_emb = pl.pallas_call(
        _ln_body,
        out_shape=jax.ShapeDtypeStruct((V, D), jnp.float32),
    )(emb, ln_g.reshape(1, D), ln_b.reshape(1, D))

    pooled_part = _sc_main(
        x.astype(jnp.int32),
        nb_x.reshape(B, LK).astype(jnp.int32),
        w_edge.reshape(B, LK).astype(jnp.int32),
        ln_emb, we[:NE - 1].reshape(NE16, 16), we[NE - 16:].reshape(16),
        eta_t.reshape(-1))

    scores = pl.pallas_call(
        _fc_body,
        out_shape=jax.ShapeDtypeStruct((B, NCLS), jnp.float32),
    )(pooled_part, fc_w, fc_b.reshape(1, NCLS))
    return scores


# speedup vs baseline: 1.2705x; 1.1376x over previous
import functools

import jax
import jax.numpy as jnp
from jax import lax
from jax.experimental import pallas as pl
from jax.experimental.pallas import tpu as pltpu
from jax.experimental.pallas import tpu_sc as plsc

B = 1024
L = 200
K = 8
D = 32
NCLS = 64
V = 10000
LK = L * K
GK = 13
LKP = GK * 128
XP = 256
NC = 2
NS = 16
NW = NC * NS
BPW = B // NW


def _ln_body(emb_ref, g_ref, b_ref, out_ref):
    e = emb_ref[...]
    m = jnp.mean(e, axis=-1, keepdims=True)
    var = jnp.mean((e - m) * (e - m), axis=-1, keepdims=True)
    out_ref[...] = ((e - m) * lax.rsqrt(var + 1e-5) * g_ref[...]
                    + b_ref[...]).astype(jnp.bfloat16)


def _fc_body(pp_ref, w_ref, b_ref, out_ref):
    pp = pp_ref[...]
    di = lax.broadcasted_iota(jnp.int32, (D * 16, D), 0)
    ei = lax.broadcasted_iota(jnp.int32, (D * 16, D), 1)
    s, l = di // 16, di % 16
    r = jnp.where(ei == 2 * ((s // 2 + l) & (D // 2 - 1)) + s % 2, 1.0, 0.0)
    p = lax.dot_general(pp, r, (((1,), (0,)), ((), ())),
                        preferred_element_type=jnp.float32)
    p = jnp.maximum(p, 1.0)
    out_ref[...] = lax.dot_general(
        p, w_ref[...], (((1,), (1,)), ((), ())),
        preferred_element_type=jnp.float32) + b_ref[...]


NE = (V - 1) * V + 1
NE16 = (NE - 1) // 16


def _sc_body(x_hbm, nb_hbm, wed_hbm, lnemb_hbm, we_hbm, wtail_hbm, eta_hbm,
             out_hbm, x_v, nb_v, wi_v, rid_v, we_v, wtail_v, eta_tbl, rows_v,
             self_v, out_v, sem):
    wid = lax.axis_index("s") * NC + lax.axis_index("c")
    b0 = wid * BPW

    iota = lax.iota(jnp.int32, 16)
    zi16 = jnp.zeros((16,), jnp.int32)
    zf16 = jnp.zeros((16,), jnp.float32)
    dcols = [(iota + dp) & 15 for dp in range(D // 2)]
    am = jnp.full((16,), -65536, jnp.int32)

    for i in range((XP - L) // 16 + 1):
        x_v[pl.ds(192 + 16 * i, 16)] = zi16
    for i in range((LKP - LK) // 16):
        nb_v[pl.ds(LK + 16 * i, 16)] = zi16
        wi_v[pl.ds(LK + 16 * i, 16)] = zi16
    pltpu.sync_copy(wtail_hbm, wtail_v)
    wt15 = plsc.load_gather(wtail_v, [iota * 0 + 15])
    pltpu.sync_copy(eta_hbm, eta_tbl.at[pl.ds(0, V)])

    def g_body(g, carry):
        tok = g * 16 + iota
        x16 = plsc.load_gather(x_v, [tok])
        m16 = jnp.where(x16 != 0, 1.0, 0.0)
        er16 = plsc.load_gather(eta_tbl, [x16])
        eta16 = 1.0 / (1.0 + jnp.exp(-er16))
        base = g * 128 + iota * 8
        cf = []
        cnt = zf16
        for k in range(K):
            idxk = base + k
            nb16 = plsc.load_gather(nb_v, [idxk])
            wi16 = plsc.load_gather(wi_v, [idxk])
            w16 = plsc.load_gather(we_v, [idxk, wi16 & 15])
            w16 = jnp.where(wi16 == NE - 1, wt15, w16)
            mk = jnp.where(nb16 != 0, 1.0, 0.0)
            cnt = cnt + mk
            cf.append(w16 * mk)
        denom = jnp.maximum(cnt, 1.0)
        a16 = m16 * (1.0 - eta16) / denom
        s16 = m16 * eta16
        cks = [cf[k] * a16 for k in range(K)]
        rws = [base + k for k in range(K)]
        for dp in range(D // 2):
            dd = dcols[dp]
            vi = plsc.load_gather(self_v, [tok, dd])
            alo = s16 * plsc.bitcast(vi << 16, jnp.float32)
            ahi = s16 * plsc.bitcast(vi & am, jnp.float32)
            for k in range(K):
                vi = plsc.load_gather(rows_v, [rws[k], dd])
                alo = alo + cks[k] * plsc.bitcast(vi << 16, jnp.float32)
                ahi = ahi + cks[k] * plsc.bitcast(vi & am, jnp.float32)
            plsc.addupdate(out_v.at[pl.ds(2 * dp * 16, 16)], alo)
            plsc.addupdate(out_v.at[pl.ds((2 * dp + 1) * 16, 16)], ahi)
        return carry

    def b_body(bi, carry):
        b = b0 + bi
        h1 = pltpu.async_copy(x_hbm.at[b], x_v.at[pl.ds(0, L)], sem)
        h2 = pltpu.async_copy(nb_hbm.at[b], nb_v.at[pl.ds(0, LK)], sem)
        h3 = pltpu.async_copy(wed_hbm.at[b], wi_v.at[pl.ds(0, LK)], sem)
        h1.wait()
        h2.wait()
        h3.wait()
        for j in range(LKP // 16):
            sl = pl.ds(j * 16, 16)
            rid_v[sl] = jnp.minimum(
                lax.shift_right_logical(wi_v[sl], 4), NE16 - 1)
        hs = [
            pltpu.async_copy(lnemb_hbm.at[nb_v], rows_v, sem),
            pltpu.async_copy(we_hbm.at[rid_v], we_v, sem),
            pltpu.async_copy(lnemb_hbm.at[x_v], self_v, sem),
        ]
        for h in hs:
            h.wait()
        for d in range(D):
            out_v[pl.ds(d * 16, 16)] = zf16
        lax.fori_loop(0, GK, g_body, 0)
        pltpu.sync_copy(out_v, out_hbm.at[b])
        return carry

    lax.fori_loop(0, BPW, b_body, 0)


_sc_main = functools.partial(
    pl.kernel,
    out_type=jax.ShapeDtypeStruct((B, D * 16), jnp.float32),
    mesh=plsc.VectorSubcoreMesh(core_axis_name="c", subcore_axis_name="s"),
    compiler_params=pltpu.CompilerParams(
        needs_layout_passes=False, use_tc_tiling_on_sc=False),
    scratch_types=[
        pltpu.VMEM((XP,), jnp.int32),
        pltpu.VMEM((LKP,), jnp.int32),
        pltpu.VMEM((LKP,), jnp.int32),
        pltpu.VMEM((LKP,), jnp.int32),
        pltpu.VMEM((LKP, 16), jnp.float32),
        pltpu.VMEM((16,), jnp.float32),
        pltpu.VMEM((V,), jnp.float32),
        pltpu.VMEM((LKP, D // 2), jnp.int32),
        pltpu.VMEM((XP, D // 2), jnp.int32),
        pltpu.VMEM((D * 16,), jnp.float32),
        pltpu.SemaphoreType.DMA,
    ],
)(_sc_body)


def kernel(x, nb_x, w_edge, emb, we, eta_t, ln_g, ln_b, fc_w, fc_b):
    ln_bf = pl.pallas_call(
        _ln_body,
        out_shape=jax.ShapeDtypeStruct((V, D), jnp.bfloat16),
    )(emb, ln_g.reshape(1, D), ln_b.reshape(1, D))
    ln_emb = lax.bitcast_convert_type(
        ln_bf.reshape(V, D // 2, 2), jnp.int32)

    pooled_part = _sc_main(
        x.astype(jnp.int32),
        nb_x.reshape(B, LK).astype(jnp.int32),
        w_edge.reshape(B, LK).astype(jnp.int32),
        ln_emb, we[:NE - 1].reshape(NE16, 16), we[NE - 16:].reshape(16),
        eta_t.reshape(-1))

    scores = pl.pallas_call(
        _fc_body,
        out_shape=jax.ShapeDtypeStruct((B, NCLS), jnp.float32),
    )(pooled_part, fc_w, fc_b.reshape(1, NCLS))
    return scores
